# Initial kernel scaffold; baseline (speedup 1.0000x reference)
import jax
import jax.numpy as jnp
from jax.experimental import pallas as pl


def kernel(x, Wg, bg, Wn, bn, Wproj, Wdown):
    raise NotImplementedError("write your pallas kernel here")


# speedup vs baseline: 1.9957x; 1.9957x over previous
import functools

import jax
import jax.numpy as jnp
from jax import lax
from jax.experimental import pallas as pl
from jax.experimental.pallas import tpu as pltpu
from jax.experimental.pallas import tpu_sc as plsc

S = 2048
D = 1024
E = 8
EP = 128
HID = 4096
CAP = 512
NSLOT = E * CAP

NC, NS = 2, 16
NW = NC * NS


def _router_body(x_ref, wg_ref, bg_ref, wn_ref, bn_ref, eps_ref,
                 slot1_ref, slot2_ref, kept1_ref, kept2_ref,
                 cg1_ref, cg2_ref, aux_ref):
    x = x_ref[...]
    logits = jnp.dot(x, wg_ref[...], preferred_element_type=jnp.float32)
    logits = logits + bg_ref[...]
    zn = jnp.dot(x, wn_ref[...], preferred_element_type=jnp.float32)
    zn = zn + bn_ref[...]
    noise = jnp.maximum(zn, 0.0) + jnp.log(1.0 + jnp.exp(-jnp.abs(zn)))
    noisy = logits + eps_ref[...] * noise

    col = lax.broadcasted_iota(jnp.int32, (S, EP), 1)
    valid = col < E
    neg = jnp.float32(-1e30)
    noisy = jnp.where(valid, noisy, neg)

    m1 = jnp.max(noisy, axis=1, keepdims=True)
    i1 = jnp.min(jnp.where(noisy == m1, col, 10**9), axis=1, keepdims=True)
    noisy2 = jnp.where(col == i1, neg, noisy)
    m2 = jnp.max(noisy2, axis=1, keepdims=True)
    i2 = jnp.min(jnp.where(noisy2 == m2, col, 10**9), axis=1, keepdims=True)

    e2 = jnp.exp(m2 - m1)
    denom = 1.0 + e2
    p1 = 1.0 / denom
    p2 = e2 / denom

    probs = jnp.where(col == i1, p1, 0.0) + jnp.where(col == i2, p2, 0.0)
    mean_e = jnp.sum(probs, axis=0, keepdims=True) * (1.0 / S)
    diff = jnp.where(col[:1, :] < E, mean_e - 1.0 / E, 0.0)
    aux_ref[...] = jnp.sum(diff * diff, keepdims=True).reshape(1, 1)

    mask = ((col == i1) | (col == i2)).astype(jnp.int32)
    c = mask
    sh = 1
    while sh < S:
        c = c + jnp.concatenate(
            [jnp.zeros((sh, EP), jnp.int32), c[:-sh, :]], axis=0)
        sh *= 2
    pos = c - mask

    pos1 = jnp.sum(jnp.where(col == i1, pos, 0), axis=1, keepdims=True)
    pos2 = jnp.sum(jnp.where(col == i2, pos, 0), axis=1, keepdims=True)
    kept1 = (pos1 < CAP).astype(jnp.int32)
    kept2 = (pos2 < CAP).astype(jnp.int32)

    slot1_ref[...] = i1 * CAP + jnp.minimum(pos1, CAP - 1)
    slot2_ref[...] = i2 * CAP + jnp.minimum(pos2, CAP - 1)
    kept1_ref[...] = kept1
    kept2_ref[...] = kept2
    lanes = jnp.zeros((S, NS), jnp.float32)
    cg1_ref[...] = lanes + p1 * kept1.astype(jnp.float32)
    cg2_ref[...] = lanes + p2 * kept2.astype(jnp.float32)


def _router(x, wg, bg, wn, bn, eps):
    out_shapes = (
        jax.ShapeDtypeStruct((S, 1), jnp.int32),
        jax.ShapeDtypeStruct((S, 1), jnp.int32),
        jax.ShapeDtypeStruct((S, 1), jnp.int32),
        jax.ShapeDtypeStruct((S, 1), jnp.int32),
        jax.ShapeDtypeStruct((S, NS), jnp.float32),
        jax.ShapeDtypeStruct((S, NS), jnp.float32),
        jax.ShapeDtypeStruct((1, 1), jnp.float32),
    )
    return pl.pallas_call(
        _router_body,
        out_shape=out_shapes,
    )(x, wg, bg, wn, bn, eps)


def _selbuild_body(s1_hbm, s2_hbm, k1_hbm, k2_hbm, sel_hbm,
                   s1v, s2v, k1v, k2v, selv):
    @pl.when((lax.axis_index("c") == 0) & (lax.axis_index("s") == 0))
    def _():
        pltpu.sync_copy(s1_hbm, s1v)
        pltpu.sync_copy(s2_hbm, s2v)
        pltpu.sync_copy(k1_hbm, k1v)
        pltpu.sync_copy(k2_hbm, k2v)

        zeros = jnp.zeros((16,), jnp.int32)

        def init(i, carry):
            selv[pl.ds(i * 16, 16)] = zeros
            return carry

        lax.fori_loop(0, NSLOT // 16, init, 0)

        def step(i, carry):
            base = i * 16
            tok = lax.iota(jnp.int32, 16) + base
            s1 = s1v[pl.ds(base, 16)]
            k1 = k1v[pl.ds(base, 16)]
            plsc.store_scatter(selv, [s1], tok, mask=k1 == 1)
            s2 = s2v[pl.ds(base, 16)]
            k2 = k2v[pl.ds(base, 16)]
            plsc.store_scatter(selv, [s2], tok, mask=k2 == 1)
            return carry

        lax.fori_loop(0, S // 16, step, 0)
        pltpu.sync_copy(selv, sel_hbm)


def _selbuild(slot1, slot2, kept1, kept2):
    mesh = plsc.VectorSubcoreMesh(core_axis_name="c", subcore_axis_name="s")
    f = functools.partial(
        pl.kernel,
        mesh=mesh,
        compiler_params=pltpu.CompilerParams(needs_layout_passes=False),
        out_type=jax.ShapeDtypeStruct((NSLOT,), jnp.int32),
        scratch_types=[
            pltpu.VMEM((S,), jnp.int32),
            pltpu.VMEM((S,), jnp.int32),
            pltpu.VMEM((S,), jnp.int32),
            pltpu.VMEM((S,), jnp.int32),
            pltpu.VMEM((NSLOT,), jnp.int32),
        ],
    )(_selbuild_body)
    return f(slot1, slot2, kept1, kept2)


def _gather_body(x_hbm, sel_hbm, xd_hbm, idxv, rowsv, sem):
    wid = lax.axis_index("s") * NC + lax.axis_index("c")
    rows_per_w = NSLOT // NW
    chunk = 64
    for c in range(rows_per_w // chunk):
        base = wid * rows_per_w + c * chunk
        pltpu.sync_copy(sel_hbm.at[pl.ds(base, chunk)], idxv)
        pltpu.async_copy(x_hbm.at[idxv], rowsv, sem).wait()
        pltpu.sync_copy(rowsv, xd_hbm.at[pl.ds(base, chunk)])


def _gather(x, sel):
    mesh = plsc.VectorSubcoreMesh(core_axis_name="c", subcore_axis_name="s")
    f = functools.partial(
        pl.kernel,
        mesh=mesh,
        compiler_params=pltpu.CompilerParams(needs_layout_passes=False),
        out_type=jax.ShapeDtypeStruct((NSLOT, D), jnp.float32),
        scratch_types=[
            pltpu.VMEM((64,), jnp.int32),
            pltpu.VMEM((64, D), jnp.float32),
            pltpu.SemaphoreType.DMA,
        ],
    )(_gather_body)
    return f(x, sel)


BH = 512


def _ffn_body(xd_ref, wp1_ref, wp2_ref, wd_ref, y_ref):
    h = pl.program_id(1)
    xd = xd_ref[...]
    w1 = wp1_ref[...].reshape(D, BH)
    w2 = wp2_ref[...].reshape(D, BH)
    wd = wd_ref[...].reshape(BH, D)
    x1 = jnp.dot(xd, w1, preferred_element_type=jnp.float32)
    x2 = jnp.dot(xd, w2, preferred_element_type=jnp.float32)
    g = x1 * (1.0 / (1.0 + jnp.exp(-x1))) * x2
    part = jnp.dot(g, wd, preferred_element_type=jnp.float32)

    @pl.when(h == 0)
    def _():
        y_ref[...] = part

    @pl.when(h > 0)
    def _():
        y_ref[...] += part


def _ffn(xd, wproj, wdown):
    nh = HID // BH
    grid = (E, nh)
    return pl.pallas_call(
        _ffn_body,
        grid=grid,
        in_specs=[
            pl.BlockSpec((CAP, D), lambda e, h: (e, 0)),
            pl.BlockSpec((1, D, BH), lambda e, h: (e, 0, h)),
            pl.BlockSpec((1, D, BH), lambda e, h: (e, 0, h + nh)),
            pl.BlockSpec((1, BH, D), lambda e, h: (e, h, 0)),
        ],
        out_specs=pl.BlockSpec((CAP, D), lambda e, h: (e, 0)),
        out_shape=jax.ShapeDtypeStruct((NSLOT, D), jnp.float32),
    )(xd, wproj, wproj, wdown)


def _combine_body(y_hbm, s1_hbm, s2_hbm, g1_hbm, g2_hbm, out_hbm,
                  i1v, i2v, g1v, g2v, ya, yb, sem):
    wid = lax.axis_index("s") * NC + lax.axis_index("c")
    tok_per_w = S // NW
    chunk = 32
    for c in range(tok_per_w // chunk):
        cb = wid * tok_per_w + c * chunk
        pltpu.sync_copy(s1_hbm.at[pl.ds(cb, chunk)], i1v)
        pltpu.sync_copy(s2_hbm.at[pl.ds(cb, chunk)], i2v)
        pltpu.sync_copy(g1_hbm.at[pl.ds(cb * NS, chunk * NS)], g1v)
        pltpu.sync_copy(g2_hbm.at[pl.ds(cb * NS, chunk * NS)], g2v)
        pltpu.async_copy(y_hbm.at[i1v], ya, sem).wait()
        pltpu.async_copy(y_hbm.at[i2v], yb, sem).wait()

        def row(r, carry):
            ga = g1v[pl.ds(pl.multiple_of(r * NS, NS), 16)]
            gb = g2v[pl.ds(pl.multiple_of(r * NS, NS), 16)]

            def col(j, carry2):
                o = pl.multiple_of(j * 64, 64)
                for u in range(4):
                    oo = o + u * 16
                    va = ya[r, pl.ds(oo, 16)]
                    vb = yb[r, pl.ds(oo, 16)]
                    ya[r, pl.ds(oo, 16)] = ga * va + gb * vb
                return carry2

            lax.fori_loop(0, D // 64, col, 0)
            return carry

        lax.fori_loop(0, chunk, row, 0)
        pltpu.sync_copy(ya, out_hbm.at[pl.ds(cb, chunk)])


def _combine(y, slot1, slot2, cg1, cg2):
    mesh = plsc.VectorSubcoreMesh(core_axis_name="c", subcore_axis_name="s")
    f = functools.partial(
        pl.kernel,
        mesh=mesh,
        compiler_params=pltpu.CompilerParams(needs_layout_passes=False),
        out_type=jax.ShapeDtypeStruct((S, D), jnp.float32),
        scratch_types=[
            pltpu.VMEM((32,), jnp.int32),
            pltpu.VMEM((32,), jnp.int32),
            pltpu.VMEM((32 * NS,), jnp.float32),
            pltpu.VMEM((32 * NS,), jnp.float32),
            pltpu.VMEM((32, D), jnp.float32),
            pltpu.VMEM((32, D), jnp.float32),
            pltpu.SemaphoreType.DMA,
        ],
    )(_combine_body)
    return f(y, slot1, slot2, cg1, cg2)


def kernel(x, Wg, bg, Wn, bn, Wproj, Wdown):
    x2 = x.reshape(S, D)
    eps = jax.random.normal(jax.random.key(42), (1, S, E),
                            dtype=jnp.float32).reshape(S, E)
    padw = ((0, 0), (0, EP - E))
    wg = jnp.pad(Wg, padw)
    wn = jnp.pad(Wn, padw)
    bgp = jnp.pad(bg, (0, EP - E)).reshape(1, EP)
    bnp = jnp.pad(bn, (0, EP - E)).reshape(1, EP)
    epsp = jnp.pad(eps, padw)

    slot1, slot2, kept1, kept2, cg1, cg2, aux = _router(
        x2, wg, bgp, wn, bnp, epsp)
    slot1 = slot1.reshape(S)
    slot2 = slot2.reshape(S)
    kept1 = kept1.reshape(S)
    kept2 = kept2.reshape(S)
    cg1 = cg1.reshape(S * NS)
    cg2 = cg2.reshape(S * NS)

    sel = _selbuild(slot1, slot2, kept1, kept2)
    xd = _gather(x2, sel)
    y = _ffn(xd, Wproj, Wdown)
    out = _combine(y, slot1, slot2, cg1, cg2)
    return out.reshape(1, S, D), aux.reshape(())


# speedup vs baseline: 2.0682x; 1.0363x over previous
import functools

import jax
import jax.numpy as jnp
from jax import lax
from jax.experimental import pallas as pl
from jax.experimental.pallas import tpu as pltpu
from jax.experimental.pallas import tpu_sc as plsc

S = 2048
D = 1024
E = 8
EP = 128
HID = 4096
CAP = 512
NSLOT = E * CAP

NC, NS = 2, 16
NW = NC * NS


def _router_body(x_ref, wg_ref, bg_ref, wn_ref, bn_ref, eps_ref,
                 slotk1_ref, slotk2_ref, slotc1_ref, slotc2_ref,
                 cg1_ref, cg2_ref, aux_ref):
    x = x_ref[...]
    logits = jnp.dot(x, wg_ref[...], preferred_element_type=jnp.float32)
    logits = logits + bg_ref[...]
    zn = jnp.dot(x, wn_ref[...], preferred_element_type=jnp.float32)
    zn = zn + bn_ref[...]
    noise = jnp.maximum(zn, 0.0) + jnp.log(1.0 + jnp.exp(-jnp.abs(zn)))
    noisy = logits + eps_ref[...] * noise

    col = lax.broadcasted_iota(jnp.int32, (S, EP), 1)
    valid = col < E
    neg = jnp.float32(-1e30)
    noisy = jnp.where(valid, noisy, neg)

    m1 = jnp.max(noisy, axis=1, keepdims=True)
    i1 = jnp.min(jnp.where(noisy == m1, col, 10**9), axis=1, keepdims=True)
    noisy2 = jnp.where(col == i1, neg, noisy)
    m2 = jnp.max(noisy2, axis=1, keepdims=True)
    i2 = jnp.min(jnp.where(noisy2 == m2, col, 10**9), axis=1, keepdims=True)

    e2 = jnp.exp(m2 - m1)
    denom = 1.0 + e2
    p1 = 1.0 / denom
    p2 = e2 / denom

    probs = jnp.where(col == i1, p1, 0.0) + jnp.where(col == i2, p2, 0.0)
    mean_e = jnp.sum(probs, axis=0, keepdims=True) * (1.0 / S)
    diff = jnp.where(col[:1, :] < E, mean_e - 1.0 / E, 0.0)
    aux_ref[...] = jnp.sum(diff * diff, keepdims=True).reshape(1, 1)

    mask = ((col == i1) | (col == i2)).astype(jnp.int32)
    c = mask
    sh = 1
    while sh < S:
        c = c + jnp.concatenate(
            [jnp.zeros((sh, EP), jnp.int32), c[:-sh, :]], axis=0)
        sh *= 2
    pos = c - mask

    pos1 = jnp.sum(jnp.where(col == i1, pos, 0), axis=1, keepdims=True)
    pos2 = jnp.sum(jnp.where(col == i2, pos, 0), axis=1, keepdims=True)
    kept1 = pos1 < CAP
    kept2 = pos2 < CAP

    slotk1_ref[...] = jnp.where(kept1, i1 * CAP + pos1, 4 * NSLOT)
    slotk2_ref[...] = jnp.where(kept2, i2 * CAP + pos2, 4 * NSLOT)
    slotc1_ref[...] = i1 * CAP + jnp.minimum(pos1, CAP - 1)
    slotc2_ref[...] = i2 * CAP + jnp.minimum(pos2, CAP - 1)
    lanes = jnp.zeros((S, NS), jnp.float32)
    cg1_ref[...] = lanes + p1 * kept1.astype(jnp.float32)
    cg2_ref[...] = lanes + p2 * kept2.astype(jnp.float32)


def _router(x, wg, bg, wn, bn, eps):
    out_shapes = (
        jax.ShapeDtypeStruct((S, 1), jnp.int32),
        jax.ShapeDtypeStruct((S, 1), jnp.int32),
        jax.ShapeDtypeStruct((S, 1), jnp.int32),
        jax.ShapeDtypeStruct((S, 1), jnp.int32),
        jax.ShapeDtypeStruct((S, NS), jnp.float32),
        jax.ShapeDtypeStruct((S, NS), jnp.float32),
        jax.ShapeDtypeStruct((1, 1), jnp.float32),
    )
    return pl.pallas_call(
        _router_body,
        out_shape=out_shapes,
    )(x, wg, bg, wn, bn, eps)


BH = 512
NH = HID // BH


def _ffn_body(x_ref, sk1_ref, sk2_ref, wp1_ref, wp2_ref, wd_ref, y_ref,
              xb_s, xe_s):
    e = pl.program_id(0)
    h = pl.program_id(1)

    @pl.when((e == 0) & (h == 0))
    def _():
        xb_s[...] = x_ref[...].astype(jnp.bfloat16)

    @pl.when(h == 0)
    def _():
        rowid = lax.broadcasted_iota(jnp.int32, (CAP, S), 0) + e * CAP
        p = ((sk1_ref[...] == rowid) | (sk2_ref[...] == rowid))
        xe = jnp.dot(p.astype(jnp.bfloat16), xb_s[...],
                     preferred_element_type=jnp.float32)
        xe_s[...] = xe.astype(jnp.bfloat16)

    xe = xe_s[...]
    w1 = wp1_ref[...].reshape(D, BH).astype(jnp.bfloat16)
    w2 = wp2_ref[...].reshape(D, BH).astype(jnp.bfloat16)
    wd = wd_ref[...].reshape(BH, D).astype(jnp.bfloat16)
    x1 = jnp.dot(xe, w1, preferred_element_type=jnp.float32)
    x2 = jnp.dot(xe, w2, preferred_element_type=jnp.float32)
    g = x1 * (1.0 / (1.0 + jnp.exp(-x1))) * x2
    part = jnp.dot(g.astype(jnp.bfloat16), wd,
                   preferred_element_type=jnp.float32)

    @pl.when(h == 0)
    def _():
        y_ref[...] = part

    @pl.when(h > 0)
    def _():
        y_ref[...] += part


def _ffn(x, slotk1, slotk2, wproj, wdown):
    grid = (E, NH)
    return pl.pallas_call(
        _ffn_body,
        grid=grid,
        in_specs=[
            pl.BlockSpec((S, D), lambda e, h: (0, 0)),
            pl.BlockSpec((1, S), lambda e, h: (0, 0)),
            pl.BlockSpec((1, S), lambda e, h: (0, 0)),
            pl.BlockSpec((1, D, BH), lambda e, h: (e, 0, h)),
            pl.BlockSpec((1, D, BH), lambda e, h: (e, 0, h + NH)),
            pl.BlockSpec((1, BH, D), lambda e, h: (e, h, 0)),
        ],
        out_specs=pl.BlockSpec((CAP, D), lambda e, h: (e, 0)),
        out_shape=jax.ShapeDtypeStruct((NSLOT, D), jnp.float32),
        scratch_shapes=[
            pltpu.VMEM((S, D), jnp.bfloat16),
            pltpu.VMEM((CAP, D), jnp.bfloat16),
        ],
    )(x, slotk1, slotk2, wproj, wproj, wdown)


def _combine_body(y_hbm, s1_hbm, s2_hbm, g1_hbm, g2_hbm, out_hbm,
                  i1v, i2v, g1v, g2v, ya, yb, sem):
    wid = lax.axis_index("s") * NC + lax.axis_index("c")
    tok_per_w = S // NW
    chunk = 32
    for c in range(tok_per_w // chunk):
        cb = wid * tok_per_w + c * chunk
        pltpu.sync_copy(s1_hbm.at[pl.ds(cb, chunk)], i1v)
        pltpu.sync_copy(s2_hbm.at[pl.ds(cb, chunk)], i2v)
        pltpu.sync_copy(g1_hbm.at[pl.ds(cb * NS, chunk * NS)], g1v)
        pltpu.sync_copy(g2_hbm.at[pl.ds(cb * NS, chunk * NS)], g2v)
        pltpu.async_copy(y_hbm.at[i1v], ya, sem).wait()
        pltpu.async_copy(y_hbm.at[i2v], yb, sem).wait()

        def row(r, carry):
            ga = g1v[pl.ds(pl.multiple_of(r * NS, NS), 16)]
            gb = g2v[pl.ds(pl.multiple_of(r * NS, NS), 16)]

            def col(j, carry2):
                o = pl.multiple_of(j * 64, 64)
                for u in range(4):
                    oo = o + u * 16
                    va = ya[r, pl.ds(oo, 16)]
                    vb = yb[r, pl.ds(oo, 16)]
                    ya[r, pl.ds(oo, 16)] = ga * va + gb * vb
                return carry2

            lax.fori_loop(0, D // 64, col, 0)
            return carry

        lax.fori_loop(0, chunk, row, 0)
        pltpu.sync_copy(ya, out_hbm.at[pl.ds(cb, chunk)])


def _combine(y, slot1, slot2, cg1, cg2):
    mesh = plsc.VectorSubcoreMesh(core_axis_name="c", subcore_axis_name="s")
    f = functools.partial(
        pl.kernel,
        mesh=mesh,
        compiler_params=pltpu.CompilerParams(needs_layout_passes=False),
        out_type=jax.ShapeDtypeStruct((S, D), jnp.float32),
        scratch_types=[
            pltpu.VMEM((32,), jnp.int32),
            pltpu.VMEM((32,), jnp.int32),
            pltpu.VMEM((32 * NS,), jnp.float32),
            pltpu.VMEM((32 * NS,), jnp.float32),
            pltpu.VMEM((32, D), jnp.float32),
            pltpu.VMEM((32, D), jnp.float32),
            pltpu.SemaphoreType.DMA,
        ],
    )(_combine_body)
    return f(y, slot1, slot2, cg1, cg2)


def kernel(x, Wg, bg, Wn, bn, Wproj, Wdown):
    x2 = x.reshape(S, D)
    eps = jax.random.normal(jax.random.key(42), (1, S, E),
                            dtype=jnp.float32).reshape(S, E)
    padw = ((0, 0), (0, EP - E))
    wg = jnp.pad(Wg, padw)
    wn = jnp.pad(Wn, padw)
    bgp = jnp.pad(bg, (0, EP - E)).reshape(1, EP)
    bnp = jnp.pad(bn, (0, EP - E)).reshape(1, EP)
    epsp = jnp.pad(eps, padw)

    slotk1, slotk2, slotc1, slotc2, cg1, cg2, aux = _router(
        x2, wg, bgp, wn, bnp, epsp)
    y = _ffn(x2, slotk1.reshape(1, S), slotk2.reshape(1, S), Wproj, Wdown)
    out = _combine(y, slotc1.reshape(S), slotc2.reshape(S),
                   cg1.reshape(S * NS), cg2.reshape(S * NS))
    return out.reshape(1, S, D), aux.reshape(())


# speedup vs baseline: 2.1298x; 1.0298x over previous
import functools

import jax
import jax.numpy as jnp
from jax import lax
from jax.experimental import pallas as pl
from jax.experimental.pallas import tpu as pltpu
from jax.experimental.pallas import tpu_sc as plsc

S = 2048
D = 1024
E = 8
EP = 128
HID = 4096
CAP = 512
NSLOT = E * CAP

NC, NS = 2, 16
NW = NC * NS

BH = 512
NH = HID // BH


def _ffn_body(x_ref, wg_ref, bg_ref, wn_ref, bn_ref, eps_ref,
              wp1_ref, wp2_ref, wd_ref,
              y_ref, slotc1_ref, slotc2_ref, cg1_ref, cg2_ref, aux_ref,
              xb_s, xe_s, sk1_s, sk2_s):
    e = pl.program_id(0)
    h = pl.program_id(1)

    @pl.when((e == 0) & (h == 0))
    def _():
        x = x_ref[...]
        xb_s[...] = x.astype(jnp.bfloat16)
        logits = jnp.dot(x, wg_ref[...], preferred_element_type=jnp.float32)
        logits = logits + bg_ref[...]
        zn = jnp.dot(x, wn_ref[...], preferred_element_type=jnp.float32)
        zn = zn + bn_ref[...]
        noise = jnp.maximum(zn, 0.0) + jnp.log(1.0 + jnp.exp(-jnp.abs(zn)))
        noisy = logits + eps_ref[...] * noise

        col = lax.broadcasted_iota(jnp.int32, (S, EP), 1)
        neg = jnp.float32(-1e30)
        noisy = jnp.where(col < E, noisy, neg)

        m1 = jnp.max(noisy, axis=1, keepdims=True)
        i1 = jnp.min(jnp.where(noisy == m1, col, 10**9), axis=1,
                     keepdims=True)
        noisy2 = jnp.where(col == i1, neg, noisy)
        m2 = jnp.max(noisy2, axis=1, keepdims=True)
        i2 = jnp.min(jnp.where(noisy2 == m2, col, 10**9), axis=1,
                     keepdims=True)

        e2 = jnp.exp(m2 - m1)
        denom = 1.0 + e2
        p1 = 1.0 / denom
        p2 = e2 / denom

        probs = jnp.where(col == i1, p1, 0.0) + jnp.where(col == i2, p2, 0.0)
        mean_e = jnp.sum(probs, axis=0, keepdims=True) * (1.0 / S)
        diff = jnp.where(col[:1, :] < E, mean_e - 1.0 / E, 0.0)
        aux_ref[...] = jnp.sum(diff * diff, keepdims=True).reshape(1, 1)

        mask = ((col == i1) | (col == i2)).astype(jnp.int32)
        c = mask
        sh = 1
        while sh < S:
            c = c + jnp.concatenate(
                [jnp.zeros((sh, EP), jnp.int32), c[:-sh, :]], axis=0)
            sh *= 2
        pos = c - mask

        pos1 = jnp.sum(jnp.where(col == i1, pos, 0), axis=1, keepdims=True)
        pos2 = jnp.sum(jnp.where(col == i2, pos, 0), axis=1, keepdims=True)
        kept1 = pos1 < CAP
        kept2 = pos2 < CAP

        sk1_s[...] = jnp.where(kept1, i1 * CAP + pos1, 4 * NSLOT).reshape(1, S)
        sk2_s[...] = jnp.where(kept2, i2 * CAP + pos2, 4 * NSLOT).reshape(1, S)
        slotc1_ref[...] = i1 * CAP + jnp.minimum(pos1, CAP - 1)
        slotc2_ref[...] = i2 * CAP + jnp.minimum(pos2, CAP - 1)
        lanes = jnp.zeros((S, NS), jnp.float32)
        cg1_ref[...] = lanes + p1 * kept1.astype(jnp.float32)
        cg2_ref[...] = lanes + p2 * kept2.astype(jnp.float32)

    @pl.when(h == 0)
    def _():
        rowid = lax.broadcasted_iota(jnp.int32, (CAP, S), 0) + e * CAP
        p = (sk1_s[...] == rowid) | (sk2_s[...] == rowid)
        xe = jnp.dot(p.astype(jnp.bfloat16), xb_s[...],
                     preferred_element_type=jnp.float32)
        xe_s[...] = xe.astype(jnp.bfloat16)

    xe = xe_s[...]
    w1 = wp1_ref[...].reshape(D, BH).astype(jnp.bfloat16)
    w2 = wp2_ref[...].reshape(D, BH).astype(jnp.bfloat16)
    wd = wd_ref[...].reshape(BH, D).astype(jnp.bfloat16)
    x1 = jnp.dot(xe, w1, preferred_element_type=jnp.float32)
    x2 = jnp.dot(xe, w2, preferred_element_type=jnp.float32)
    g = x1 * (1.0 / (1.0 + jnp.exp(-x1))) * x2
    part = jnp.dot(g.astype(jnp.bfloat16), wd,
                   preferred_element_type=jnp.float32)

    @pl.when(h == 0)
    def _():
        y_ref[...] = part

    @pl.when(h > 0)
    def _():
        y_ref[...] += part


def _ffn(x, wg, bg, wn, bn, eps, wproj, wdown):
    grid = (E, NH)
    zero2 = lambda e, h: (0, 0)
    out_shapes = (
        jax.ShapeDtypeStruct((NSLOT, D), jnp.float32),
        jax.ShapeDtypeStruct((S, 1), jnp.int32),
        jax.ShapeDtypeStruct((S, 1), jnp.int32),
        jax.ShapeDtypeStruct((S, NS), jnp.float32),
        jax.ShapeDtypeStruct((S, NS), jnp.float32),
        jax.ShapeDtypeStruct((1, 1), jnp.float32),
    )
    return pl.pallas_call(
        _ffn_body,
        grid=grid,
        in_specs=[
            pl.BlockSpec((S, D), zero2),
            pl.BlockSpec((D, EP), zero2),
            pl.BlockSpec((1, EP), zero2),
            pl.BlockSpec((D, EP), zero2),
            pl.BlockSpec((1, EP), zero2),
            pl.BlockSpec((S, EP), zero2),
            pl.BlockSpec((1, D, BH), lambda e, h: (e, 0, h)),
            pl.BlockSpec((1, D, BH), lambda e, h: (e, 0, h + NH)),
            pl.BlockSpec((1, BH, D), lambda e, h: (e, h, 0)),
        ],
        out_specs=(
            pl.BlockSpec((CAP, D), lambda e, h: (e, 0)),
            pl.BlockSpec((S, 1), zero2),
            pl.BlockSpec((S, 1), zero2),
            pl.BlockSpec((S, NS), zero2),
            pl.BlockSpec((S, NS), zero2),
            pl.BlockSpec((1, 1), zero2),
        ),
        out_shape=out_shapes,
        scratch_shapes=[
            pltpu.VMEM((S, D), jnp.bfloat16),
            pltpu.VMEM((CAP, D), jnp.bfloat16),
            pltpu.VMEM((1, S), jnp.int32),
            pltpu.VMEM((1, S), jnp.int32),
        ],
    )(x, wg, bg, wn, bn, eps, wproj, wproj, wdown)


def _combine_body(y_hbm, s1_hbm, s2_hbm, g1_hbm, g2_hbm, out_hbm,
                  i1v, i2v, g1v, g2v, ya, yb, sema, semb):
    wid = lax.axis_index("s") * NC + lax.axis_index("c")
    tok_per_w = S // NW
    chunk = 32
    for c in range(tok_per_w // chunk):
        cb = wid * tok_per_w + c * chunk
        pltpu.sync_copy(s1_hbm.at[pl.ds(cb, chunk)], i1v)
        pltpu.sync_copy(s2_hbm.at[pl.ds(cb, chunk)], i2v)
        pltpu.sync_copy(g1_hbm.at[pl.ds(cb * NS, chunk * NS)], g1v)
        pltpu.sync_copy(g2_hbm.at[pl.ds(cb * NS, chunk * NS)], g2v)
        cpa = pltpu.async_copy(y_hbm.at[i1v], ya, sema)
        cpb = pltpu.async_copy(y_hbm.at[i2v], yb, semb)
        cpa.wait()
        cpb.wait()

        def row(r, carry):
            ga = g1v[pl.ds(pl.multiple_of(r * NS, NS), 16)]
            gb = g2v[pl.ds(pl.multiple_of(r * NS, NS), 16)]

            def col(j, carry2):
                o = pl.multiple_of(j * 64, 64)
                for u in range(4):
                    oo = o + u * 16
                    va = ya[r, pl.ds(oo, 16)]
                    vb = yb[r, pl.ds(oo, 16)]
                    ya[r, pl.ds(oo, 16)] = ga * va + gb * vb
                return carry2

            lax.fori_loop(0, D // 64, col, 0)
            return carry

        lax.fori_loop(0, chunk, row, 0)
        pltpu.sync_copy(ya, out_hbm.at[pl.ds(cb, chunk)])


def _combine(y, slot1, slot2, cg1, cg2):
    mesh = plsc.VectorSubcoreMesh(core_axis_name="c", subcore_axis_name="s")
    f = functools.partial(
        pl.kernel,
        mesh=mesh,
        compiler_params=pltpu.CompilerParams(needs_layout_passes=False),
        out_type=jax.ShapeDtypeStruct((S, D), jnp.float32),
        scratch_types=[
            pltpu.VMEM((32,), jnp.int32),
            pltpu.VMEM((32,), jnp.int32),
            pltpu.VMEM((32 * NS,), jnp.float32),
            pltpu.VMEM((32 * NS,), jnp.float32),
            pltpu.VMEM((32, D), jnp.float32),
            pltpu.VMEM((32, D), jnp.float32),
            pltpu.SemaphoreType.DMA,
            pltpu.SemaphoreType.DMA,
        ],
    )(_combine_body)
    return f(y, slot1, slot2, cg1, cg2)


def kernel(x, Wg, bg, Wn, bn, Wproj, Wdown):
    x2 = x.reshape(S, D)
    eps = jax.random.normal(jax.random.key(42), (1, S, E),
                            dtype=jnp.float32).reshape(S, E)
    padw = ((0, 0), (0, EP - E))
    wg = jnp.pad(Wg, padw)
    wn = jnp.pad(Wn, padw)
    bgp = jnp.pad(bg, (0, EP - E)).reshape(1, EP)
    bnp = jnp.pad(bn, (0, EP - E)).reshape(1, EP)
    epsp = jnp.pad(eps, padw)

    y, slotc1, slotc2, cg1, cg2, aux = _ffn(
        x2, wg, bgp, wn, bnp, epsp, Wproj, Wdown)
    out = _combine(y, slotc1.reshape(S), slotc2.reshape(S),
                   cg1.reshape(S * NS), cg2.reshape(S * NS))
    return out.reshape(1, S, D), aux.reshape(())


# speedup vs baseline: 2.3253x; 1.0918x over previous
import functools

import jax
import jax.numpy as jnp
from jax import lax
from jax.experimental import pallas as pl
from jax.experimental.pallas import tpu as pltpu
from jax.experimental.pallas import tpu_sc as plsc

S = 2048
D = 1024
E = 8
EP = 128
HID = 4096
CAP = 512
NSLOT = E * CAP

NC, NS = 2, 16
NW = NC * NS

BH = 1024
NH = HID // BH


def _ffn_body(x_ref, wg_ref, bg_ref, wn_ref, bn_ref, eps_ref,
              wp1_ref, wp2_ref, wd_ref,
              y_ref, slotc1_ref, slotc2_ref, cg1_ref, cg2_ref, aux_ref,
              xb_s, xe_s, sk1_s, sk2_s):
    e = pl.program_id(0)
    h = pl.program_id(1)

    @pl.when((e == 0) & (h == 0))
    def _():
        x = x_ref[...]
        xb_s[...] = x.astype(jnp.bfloat16)
        logits = jnp.dot(x, wg_ref[...], preferred_element_type=jnp.float32)
        logits = logits + bg_ref[...]
        zn = jnp.dot(x, wn_ref[...], preferred_element_type=jnp.float32)
        zn = zn + bn_ref[...]
        noise = jnp.maximum(zn, 0.0) + jnp.log(1.0 + jnp.exp(-jnp.abs(zn)))
        noisy = logits + eps_ref[...] * noise

        col = lax.broadcasted_iota(jnp.int32, (S, EP), 1)
        neg = jnp.float32(-1e30)
        noisy = jnp.where(col < E, noisy, neg)

        m1 = jnp.max(noisy, axis=1, keepdims=True)
        i1 = jnp.min(jnp.where(noisy == m1, col, 10**9), axis=1,
                     keepdims=True)
        noisy2 = jnp.where(col == i1, neg, noisy)
        m2 = jnp.max(noisy2, axis=1, keepdims=True)
        i2 = jnp.min(jnp.where(noisy2 == m2, col, 10**9), axis=1,
                     keepdims=True)

        e2 = jnp.exp(m2 - m1)
        denom = 1.0 + e2
        p1 = 1.0 / denom
        p2 = e2 / denom

        probs = jnp.where(col == i1, p1, 0.0) + jnp.where(col == i2, p2, 0.0)
        mean_e = jnp.sum(probs, axis=0, keepdims=True) * (1.0 / S)
        diff = jnp.where(col[:1, :] < E, mean_e - 1.0 / E, 0.0)
        aux_ref[...] = jnp.sum(diff * diff, keepdims=True).reshape(1, 1)

        mask = ((col == i1) | (col == i2)).astype(jnp.int32)
        c = mask
        sh = 1
        while sh < S:
            c = c + jnp.concatenate(
                [jnp.zeros((sh, EP), jnp.int32), c[:-sh, :]], axis=0)
            sh *= 2
        pos = c - mask

        pos1 = jnp.sum(jnp.where(col == i1, pos, 0), axis=1, keepdims=True)
        pos2 = jnp.sum(jnp.where(col == i2, pos, 0), axis=1, keepdims=True)
        kept1 = pos1 < CAP
        kept2 = pos2 < CAP

        sk1_s[...] = jnp.where(kept1, i1 * CAP + pos1, 4 * NSLOT).reshape(1, S)
        sk2_s[...] = jnp.where(kept2, i2 * CAP + pos2, 4 * NSLOT).reshape(1, S)
        slotc1_ref[...] = i1 * CAP + jnp.minimum(pos1, CAP - 1)
        slotc2_ref[...] = i2 * CAP + jnp.minimum(pos2, CAP - 1)
        lanes = jnp.zeros((S, NS), jnp.float32)
        cg1_ref[...] = lanes + p1 * kept1.astype(jnp.float32)
        cg2_ref[...] = lanes + p2 * kept2.astype(jnp.float32)

    @pl.when(h == 0)
    def _():
        rowid = lax.broadcasted_iota(jnp.int32, (CAP, S), 0) + e * CAP
        p = (sk1_s[...] == rowid) | (sk2_s[...] == rowid)
        xe = jnp.dot(p.astype(jnp.bfloat16), xb_s[...],
                     preferred_element_type=jnp.float32)
        xe_s[...] = xe.astype(jnp.bfloat16)

    xe = xe_s[...]
    w1 = wp1_ref[...].reshape(D, BH).astype(jnp.bfloat16)
    w2 = wp2_ref[...].reshape(D, BH).astype(jnp.bfloat16)
    wd = wd_ref[...].reshape(BH, D).astype(jnp.bfloat16)
    x1 = jnp.dot(xe, w1, preferred_element_type=jnp.float32)
    x2 = jnp.dot(xe, w2, preferred_element_type=jnp.float32)
    g = x1 * (1.0 / (1.0 + jnp.exp(-x1))) * x2
    part = jnp.dot(g.astype(jnp.bfloat16), wd,
                   preferred_element_type=jnp.float32)

    @pl.when(h == 0)
    def _():
        y_ref[...] = part

    @pl.when(h > 0)
    def _():
        y_ref[...] += part


def _ffn(x, wg, bg, wn, bn, eps, wproj, wdown):
    grid = (E, NH)
    zero2 = lambda e, h: (0, 0)
    out_shapes = (
        jax.ShapeDtypeStruct((NSLOT, D), jnp.float32),
        jax.ShapeDtypeStruct((S, 1), jnp.int32),
        jax.ShapeDtypeStruct((S, 1), jnp.int32),
        jax.ShapeDtypeStruct((S, NS), jnp.float32),
        jax.ShapeDtypeStruct((S, NS), jnp.float32),
        jax.ShapeDtypeStruct((1, 1), jnp.float32),
    )
    return pl.pallas_call(
        _ffn_body,
        grid=grid,
        in_specs=[
            pl.BlockSpec((S, D), zero2),
            pl.BlockSpec((D, EP), zero2),
            pl.BlockSpec((1, EP), zero2),
            pl.BlockSpec((D, EP), zero2),
            pl.BlockSpec((1, EP), zero2),
            pl.BlockSpec((S, EP), zero2),
            pl.BlockSpec((1, D, BH), lambda e, h: (e, 0, h)),
            pl.BlockSpec((1, D, BH), lambda e, h: (e, 0, h + NH)),
            pl.BlockSpec((1, BH, D), lambda e, h: (e, h, 0)),
        ],
        out_specs=(
            pl.BlockSpec((CAP, D), lambda e, h: (e, 0)),
            pl.BlockSpec((S, 1), zero2),
            pl.BlockSpec((S, 1), zero2),
            pl.BlockSpec((S, NS), zero2),
            pl.BlockSpec((S, NS), zero2),
            pl.BlockSpec((1, 1), zero2),
        ),
        out_shape=out_shapes,
        scratch_shapes=[
            pltpu.VMEM((S, D), jnp.bfloat16),
            pltpu.VMEM((CAP, D), jnp.bfloat16),
            pltpu.VMEM((1, S), jnp.int32),
            pltpu.VMEM((1, S), jnp.int32),
        ],
    )(x, wg, bg, wn, bn, eps, wproj, wproj, wdown)


def _combine_body(y_hbm, s1_hbm, s2_hbm, g1_hbm, g2_hbm, out_hbm,
                  i1v, i2v, g1v, g2v, ya, yb, sema, semb):
    wid = lax.axis_index("s") * NC + lax.axis_index("c")
    tok_per_w = S // NW
    chunk = 32
    for c in range(tok_per_w // chunk):
        cb = wid * tok_per_w + c * chunk
        pltpu.sync_copy(s1_hbm.at[pl.ds(cb, chunk)], i1v)
        pltpu.sync_copy(s2_hbm.at[pl.ds(cb, chunk)], i2v)
        pltpu.sync_copy(g1_hbm.at[pl.ds(cb * NS, chunk * NS)], g1v)
        pltpu.sync_copy(g2_hbm.at[pl.ds(cb * NS, chunk * NS)], g2v)
        cpa = pltpu.async_copy(y_hbm.at[i1v], ya, sema)
        cpb = pltpu.async_copy(y_hbm.at[i2v], yb, semb)
        cpa.wait()
        cpb.wait()

        def row(r, carry):
            ga = g1v[pl.ds(pl.multiple_of(r * NS, NS), 16)]
            gb = g2v[pl.ds(pl.multiple_of(r * NS, NS), 16)]

            def col(j, carry2):
                o = pl.multiple_of(j * 64, 64)
                for u in range(4):
                    oo = o + u * 16
                    va = ya[r, pl.ds(oo, 16)]
                    vb = yb[r, pl.ds(oo, 16)]
                    ya[r, pl.ds(oo, 16)] = ga * va + gb * vb
                return carry2

            lax.fori_loop(0, D // 64, col, 0)
            return carry

        lax.fori_loop(0, chunk, row, 0)
        pltpu.sync_copy(ya, out_hbm.at[pl.ds(cb, chunk)])


def _combine(y, slot1, slot2, cg1, cg2):
    mesh = plsc.VectorSubcoreMesh(core_axis_name="c", subcore_axis_name="s")
    f = functools.partial(
        pl.kernel,
        mesh=mesh,
        compiler_params=pltpu.CompilerParams(needs_layout_passes=False),
        out_type=jax.ShapeDtypeStruct((S, D), jnp.float32),
        scratch_types=[
            pltpu.VMEM((32,), jnp.int32),
            pltpu.VMEM((32,), jnp.int32),
            pltpu.VMEM((32 * NS,), jnp.float32),
            pltpu.VMEM((32 * NS,), jnp.float32),
            pltpu.VMEM((32, D), jnp.float32),
            pltpu.VMEM((32, D), jnp.float32),
            pltpu.SemaphoreType.DMA,
            pltpu.SemaphoreType.DMA,
        ],
    )(_combine_body)
    return f(y, slot1, slot2, cg1, cg2)


def kernel(x, Wg, bg, Wn, bn, Wproj, Wdown):
    x2 = x.reshape(S, D)
    eps = jax.random.normal(jax.random.key(42), (1, S, E),
                            dtype=jnp.float32).reshape(S, E)
    padw = ((0, 0), (0, EP - E))
    wg = jnp.pad(Wg, padw)
    wn = jnp.pad(Wn, padw)
    bgp = jnp.pad(bg, (0, EP - E)).reshape(1, EP)
    bnp = jnp.pad(bn, (0, EP - E)).reshape(1, EP)
    epsp = jnp.pad(eps, padw)

    y, slotc1, slotc2, cg1, cg2, aux = _ffn(
        x2, wg, bgp, wn, bnp, epsp, Wproj, Wdown)
    out = _combine(y, slotc1.reshape(S), slotc2.reshape(S),
                   cg1.reshape(S * NS), cg2.reshape(S * NS))
    return out.reshape(1, S, D), aux.reshape(())


# speedup vs baseline: 2.3684x; 1.0185x over previous
import functools

import jax
import jax.numpy as jnp
from jax import lax
from jax.experimental import pallas as pl
from jax.experimental.pallas import tpu as pltpu
from jax.experimental.pallas import tpu_sc as plsc

S = 2048
D = 1024
E = 8
EP = 128
HID = 4096
CAP = 512
NSLOT = E * CAP

NC, NS = 2, 16
NW = NC * NS

BH = 1024
NH = HID // BH


def _ffn_body(x_ref, wg_ref, bg_ref, wn_ref, bn_ref, eps_ref,
              wp1_ref, wp2_ref, wd_ref,
              y_ref, slotc1_ref, slotc2_ref, cg1_ref, cg2_ref, aux_ref,
              xb_s, xe_s, sk1_s, sk2_s):
    e = pl.program_id(0)
    h = pl.program_id(1)

    @pl.when((e == 0) & (h == 0))
    def _():
        x = x_ref[...]
        xb_s[...] = x.astype(jnp.bfloat16)
        logits = jnp.dot(x, wg_ref[...], preferred_element_type=jnp.float32)
        logits = logits + bg_ref[...]
        zn = jnp.dot(x, wn_ref[...], preferred_element_type=jnp.float32)
        zn = zn + bn_ref[...]
        noise = jnp.maximum(zn, 0.0) + jnp.log(1.0 + jnp.exp(-jnp.abs(zn)))
        noisy = logits + eps_ref[...] * noise

        col = lax.broadcasted_iota(jnp.int32, (S, EP), 1)
        neg = jnp.float32(-1e30)
        noisy = jnp.where(col < E, noisy, neg)

        m1 = jnp.max(noisy, axis=1, keepdims=True)
        i1 = jnp.min(jnp.where(noisy == m1, col, 10**9), axis=1,
                     keepdims=True)
        noisy2 = jnp.where(col == i1, neg, noisy)
        m2 = jnp.max(noisy2, axis=1, keepdims=True)
        i2 = jnp.min(jnp.where(noisy2 == m2, col, 10**9), axis=1,
                     keepdims=True)

        e2 = jnp.exp(m2 - m1)
        denom = 1.0 + e2
        p1 = 1.0 / denom
        p2 = e2 / denom

        probs = jnp.where(col == i1, p1, 0.0) + jnp.where(col == i2, p2, 0.0)
        mean_e = jnp.sum(probs, axis=0, keepdims=True) * (1.0 / S)
        diff = jnp.where(col[:1, :] < E, mean_e - 1.0 / E, 0.0)
        aux_ref[...] = jnp.sum(diff * diff, keepdims=True).reshape(1, 1)

        mask = ((col == i1) | (col == i2)).astype(jnp.int32)
        c = mask
        sh = 1
        while sh < S:
            c = c + jnp.concatenate(
                [jnp.zeros((sh, EP), jnp.int32), c[:-sh, :]], axis=0)
            sh *= 2
        pos = c - mask

        pos1 = jnp.sum(jnp.where(col == i1, pos, 0), axis=1, keepdims=True)
        pos2 = jnp.sum(jnp.where(col == i2, pos, 0), axis=1, keepdims=True)
        kept1 = pos1 < CAP
        kept2 = pos2 < CAP

        sk1_s[...] = jnp.where(kept1, i1 * CAP + pos1, 4 * NSLOT).reshape(1, S)
        sk2_s[...] = jnp.where(kept2, i2 * CAP + pos2, 4 * NSLOT).reshape(1, S)
        slotc1_ref[...] = i1 * CAP + jnp.minimum(pos1, CAP - 1)
        slotc2_ref[...] = i2 * CAP + jnp.minimum(pos2, CAP - 1)
        lanes = jnp.zeros((S, NS), jnp.float32)
        cg1_ref[...] = lanes + p1 * kept1.astype(jnp.float32)
        cg2_ref[...] = lanes + p2 * kept2.astype(jnp.float32)

    @pl.when(h == 0)
    def _():
        rowid = lax.broadcasted_iota(jnp.int32, (CAP, S), 0) + e * CAP
        p = (sk1_s[...] == rowid) | (sk2_s[...] == rowid)
        xe = jnp.dot(p.astype(jnp.bfloat16), xb_s[...],
                     preferred_element_type=jnp.float32)
        xe_s[...] = xe.astype(jnp.bfloat16)

    xe = xe_s[...]
    w1 = wp1_ref[...].reshape(D, BH).astype(jnp.bfloat16)
    w2 = wp2_ref[...].reshape(D, BH).astype(jnp.bfloat16)
    wd = wd_ref[...].reshape(BH, D).astype(jnp.bfloat16)
    x1 = jnp.dot(xe, w1, preferred_element_type=jnp.float32)
    x2 = jnp.dot(xe, w2, preferred_element_type=jnp.float32)
    g = x1 * (1.0 / (1.0 + jnp.exp(-x1))) * x2
    part = jnp.dot(g.astype(jnp.bfloat16), wd,
                   preferred_element_type=jnp.float32)

    @pl.when(h == 0)
    def _():
        y_ref[...] = part

    @pl.when(h > 0)
    def _():
        y_ref[...] += part


def _ffn(x, wg, bg, wn, bn, eps, wproj, wdown):
    grid = (E, NH)
    zero2 = lambda e, h: (0, 0)
    out_shapes = (
        jax.ShapeDtypeStruct((NSLOT, D), jnp.float32),
        jax.ShapeDtypeStruct((S, 1), jnp.int32),
        jax.ShapeDtypeStruct((S, 1), jnp.int32),
        jax.ShapeDtypeStruct((S, NS), jnp.float32),
        jax.ShapeDtypeStruct((S, NS), jnp.float32),
        jax.ShapeDtypeStruct((1, 1), jnp.float32),
    )
    return pl.pallas_call(
        _ffn_body,
        grid=grid,
        in_specs=[
            pl.BlockSpec((S, D), zero2),
            pl.BlockSpec((D, EP), zero2),
            pl.BlockSpec((1, EP), zero2),
            pl.BlockSpec((D, EP), zero2),
            pl.BlockSpec((1, EP), zero2),
            pl.BlockSpec((S, EP), zero2),
            pl.BlockSpec((1, D, BH), lambda e, h: (e, 0, h)),
            pl.BlockSpec((1, D, BH), lambda e, h: (e, 0, h + NH)),
            pl.BlockSpec((1, BH, D), lambda e, h: (e, h, 0)),
        ],
        out_specs=(
            pl.BlockSpec((CAP, D), lambda e, h: (e, 0)),
            pl.BlockSpec((S, 1), zero2),
            pl.BlockSpec((S, 1), zero2),
            pl.BlockSpec((S, NS), zero2),
            pl.BlockSpec((S, NS), zero2),
            pl.BlockSpec((1, 1), zero2),
        ),
        out_shape=out_shapes,
        scratch_shapes=[
            pltpu.VMEM((S, D), jnp.bfloat16),
            pltpu.VMEM((CAP, D), jnp.bfloat16),
            pltpu.VMEM((1, S), jnp.int32),
            pltpu.VMEM((1, S), jnp.int32),
        ],
    )(x, wg, bg, wn, bn, eps, wproj, wproj, wdown)


def _combine_body(y_hbm, s1_hbm, s2_hbm, g1_hbm, g2_hbm, out_hbm,
                  i1v0, i2v0, i1v1, i2v1, g1v, g2v,
                  ya0, yb0, ya1, yb1, sa0, sb0, sa1, sb1):
    wid = lax.axis_index("s") * NC + lax.axis_index("c")
    tok_per_w = S // NW
    chunk = 16
    nch = tok_per_w // chunk
    base = wid * tok_per_w
    pltpu.sync_copy(g1_hbm.at[pl.ds(base * NS, tok_per_w * NS)], g1v)
    pltpu.sync_copy(g2_hbm.at[pl.ds(base * NS, tok_per_w * NS)], g2v)

    idx = [(i1v0, i2v0), (i1v1, i2v1)]
    buf = [(ya0, yb0), (ya1, yb1)]
    sem = [(sa0, sb0), (sa1, sb1)]

    def issue(c):
        par = c % 2
        cb = base + c * chunk
        i1, i2 = idx[par]
        ba, bb = buf[par]
        ma, mb = sem[par]
        pltpu.sync_copy(s1_hbm.at[pl.ds(cb, chunk)], i1)
        pltpu.sync_copy(s2_hbm.at[pl.ds(cb, chunk)], i2)
        return (pltpu.async_copy(y_hbm.at[i1], ba, ma),
                pltpu.async_copy(y_hbm.at[i2], bb, mb))

    pend = [issue(0), issue(1)]
    for c in range(nch):
        par = c % 2
        cpa, cpb = pend[par]
        cpa.wait()
        cpb.wait()
        ba, bb = buf[par]

        def row(r, carry):
            go = pl.multiple_of((c * chunk + r) * NS, NS)
            ga = g1v[pl.ds(go, 16)]
            gb = g2v[pl.ds(go, 16)]

            def col(j, carry2):
                o = pl.multiple_of(j * 64, 64)
                for u in range(4):
                    oo = o + u * 16
                    va = ba[r, pl.ds(oo, 16)]
                    vb = bb[r, pl.ds(oo, 16)]
                    ba[r, pl.ds(oo, 16)] = ga * va + gb * vb
                return carry2

            lax.fori_loop(0, D // 64, col, 0)
            return carry

        lax.fori_loop(0, chunk, row, 0)
        pltpu.sync_copy(ba, out_hbm.at[pl.ds(base + c * chunk, chunk)])
        if c + 2 < nch:
            pend[par] = issue(c + 2)


def _combine(y, slot1, slot2, cg1, cg2):
    mesh = plsc.VectorSubcoreMesh(core_axis_name="c", subcore_axis_name="s")
    f = functools.partial(
        pl.kernel,
        mesh=mesh,
        compiler_params=pltpu.CompilerParams(needs_layout_passes=False),
        out_type=jax.ShapeDtypeStruct((S, D), jnp.float32),
        scratch_types=[
            pltpu.VMEM((16,), jnp.int32),
            pltpu.VMEM((16,), jnp.int32),
            pltpu.VMEM((16,), jnp.int32),
            pltpu.VMEM((16,), jnp.int32),
            pltpu.VMEM((64 * NS,), jnp.float32),
            pltpu.VMEM((64 * NS,), jnp.float32),
            pltpu.VMEM((16, D), jnp.float32),
            pltpu.VMEM((16, D), jnp.float32),
            pltpu.VMEM((16, D), jnp.float32),
            pltpu.VMEM((16, D), jnp.float32),
            pltpu.SemaphoreType.DMA,
            pltpu.SemaphoreType.DMA,
            pltpu.SemaphoreType.DMA,
            pltpu.SemaphoreType.DMA,
        ],
    )(_combine_body)
    return f(y, slot1, slot2, cg1, cg2)


def kernel(x, Wg, bg, Wn, bn, Wproj, Wdown):
    x2 = x.reshape(S, D)
    eps = jax.random.normal(jax.random.key(42), (1, S, E),
                            dtype=jnp.float32).reshape(S, E)
    padw = ((0, 0), (0, EP - E))
    wg = jnp.pad(Wg, padw)
    wn = jnp.pad(Wn, padw)
    bgp = jnp.pad(bg, (0, EP - E)).reshape(1, EP)
    bnp = jnp.pad(bn, (0, EP - E)).reshape(1, EP)
    epsp = jnp.pad(eps, padw)

    y, slotc1, slotc2, cg1, cg2, aux = _ffn(
        x2, wg, bgp, wn, bnp, epsp, Wproj, Wdown)
    out = _combine(y, slotc1.reshape(S), slotc2.reshape(S),
                   cg1.reshape(S * NS), cg2.reshape(S * NS))
    return out.reshape(1, S, D), aux.reshape(())
